# Initial kernel scaffold; baseline (speedup 1.0000x reference)
import jax
import jax.numpy as jnp
from jax.experimental import pallas as pl


def kernel(hidden_states, W_gate, W_gate_up, W_down, num_global_tokens, max_num_tokens_per_gpu):
    raise NotImplementedError("write your pallas kernel here")


# speedup vs baseline: 1.5970x; 1.5970x over previous
import jax
import jax.numpy as jnp
from jax.experimental import pallas as pl
from jax.experimental.pallas import tpu as pltpu

T = 2048
D = 768
E = 64
K = 2
F = 512

TM = 128
N_TILES = (T * K) // TM + (E - 1)
NG = N_TILES * TM


def _router_kernel(hs_ref, wg_ref, idx_ref, w_ref):
    logits = jnp.dot(hs_ref[...], wg_ref[...], preferred_element_type=jnp.float32)
    m = jnp.max(logits, axis=1, keepdims=True)
    p = jnp.exp(logits - m)
    p = p / jnp.sum(p, axis=1, keepdims=True)
    iota = jax.lax.broadcasted_iota(jnp.int32, (T, E), 1)
    m1 = jnp.max(p, axis=1, keepdims=True)
    i1 = jnp.min(jnp.where(p == m1, iota, E), axis=1, keepdims=True)
    p2 = jnp.where(iota == i1, -1e30, p)
    m2 = jnp.max(p2, axis=1, keepdims=True)
    i2 = jnp.min(jnp.where(p2 == m2, iota, E), axis=1, keepdims=True)
    s = m1 + m2
    idx_ref[...] = jnp.concatenate([i1, i2], axis=1)
    w_ref[...] = jnp.concatenate([m1 / s, m2 / s], axis=1)


def _moe_kernel(te_ref, tok_ref, wt_ref, hs_ref, wgu_ref, wd_ref, out_ref,
                x_s, y_s):
    i = pl.program_id(0)

    @pl.when(i == 0)
    def _():
        out_ref[...] = jnp.zeros_like(out_ref)

    base = i * TM

    def gather(r, carry):
        w = wt_ref[base + r]

        @pl.when(w > 0.0)
        def _():
            t = tok_ref[base + r]
            x_s[pl.ds(r, 1), :] = hs_ref[pl.ds(t, 1), :]
        return carry

    jax.lax.fori_loop(0, TM, gather, 0)

    x = x_s[...]
    gu = jnp.dot(x, wgu_ref[0], preferred_element_type=jnp.float32)
    g = gu[:, :F]
    u = gu[:, F:]
    h = g * jax.nn.sigmoid(g) * u
    y_s[...] = jnp.dot(h, wd_ref[0], preferred_element_type=jnp.float32)

    def scatter(r, carry):
        w = wt_ref[base + r]

        @pl.when(w > 0.0)
        def _():
            t = tok_ref[base + r]
            out_ref[pl.ds(t, 1), :] += w * y_s[pl.ds(r, 1), :]
        return carry

    jax.lax.fori_loop(0, TM, scatter, 0)


def kernel(hidden_states, W_gate, W_gate_up, W_down, num_global_tokens,
           max_num_tokens_per_gpu):
    hs = hidden_states
    topk_idx, topk_w = pl.pallas_call(
        _router_kernel,
        out_shape=(
            jax.ShapeDtypeStruct((T, K), jnp.int32),
            jax.ShapeDtypeStruct((T, K), jnp.float32),
        ),
    )(hs, W_gate)

    e_flat = topk_idx.reshape(-1)
    w_flat = topk_w.reshape(-1)
    t_flat = jnp.repeat(jnp.arange(T, dtype=jnp.int32), K)
    onehot = e_flat[:, None] == jnp.arange(E, dtype=jnp.int32)[None, :]
    csum = jnp.cumsum(onehot.astype(jnp.int32), axis=0)
    counts = csum[-1]
    rank = jnp.sum(jnp.where(onehot, csum - 1, 0), axis=1)
    padded = ((counts + TM - 1) // TM) * TM
    pad_end = jnp.cumsum(padded)
    pad_start = pad_end - padded
    slot = pad_start[e_flat] + rank
    row_token = jnp.zeros((NG,), jnp.int32).at[slot].set(t_flat)
    row_weight = jnp.zeros((NG,), jnp.float32).at[slot].set(w_flat)
    total = pad_end[-1]
    tile_start = jnp.arange(N_TILES, dtype=jnp.int32) * TM
    te = jnp.searchsorted(pad_end, jnp.minimum(tile_start, total - 1),
                          side="right").astype(jnp.int32)
    tile_expert = jnp.minimum(te, E - 1)

    grid_spec = pltpu.PrefetchScalarGridSpec(
        num_scalar_prefetch=3,
        grid=(N_TILES,),
        in_specs=[
            pl.BlockSpec((T, D), lambda i, te, tok, w: (0, 0)),
            pl.BlockSpec((1, D, 2 * F), lambda i, te, tok, w: (te[i], 0, 0)),
            pl.BlockSpec((1, F, D), lambda i, te, tok, w: (te[i], 0, 0)),
        ],
        out_specs=pl.BlockSpec((T, D), lambda i, te, tok, w: (0, 0)),
        scratch_shapes=[
            pltpu.VMEM((TM, D), jnp.float32),
            pltpu.VMEM((TM, D), jnp.float32),
        ],
    )
    out = pl.pallas_call(
        _moe_kernel,
        grid_spec=grid_spec,
        out_shape=jax.ShapeDtypeStruct((T, D), jnp.float32),
    )(tile_expert, row_token, row_weight, hs, W_gate_up, W_down)
    return out


# speedup vs baseline: 3.3876x; 2.1212x over previous
import functools

import jax
import jax.numpy as jnp
from jax import lax
from jax.experimental import pallas as pl
from jax.experimental.pallas import tpu as pltpu
from jax.experimental.pallas import tpu_sc as plsc

T = 2048
D = 768
E = 64
K = 2
F = 512

TM = 128
N_TILES = (T * K) // TM + (E - 1)
NG = N_TILES * TM

NW = 32
GB = (T * K) // NW
CB = T // NW
LANES = 16
DC = D // LANES


def _router_kernel(hs_ref, wg_ref, idx_ref, w_ref):
    logits = jnp.dot(hs_ref[...], wg_ref[...], preferred_element_type=jnp.float32)
    m = jnp.max(logits, axis=1, keepdims=True)
    p = jnp.exp(logits - m)
    p = p / jnp.sum(p, axis=1, keepdims=True)
    iota = jax.lax.broadcasted_iota(jnp.int32, (T, E), 1)
    m1 = jnp.max(p, axis=1, keepdims=True)
    i1 = jnp.min(jnp.where(p == m1, iota, E), axis=1, keepdims=True)
    p2 = jnp.where(iota == i1, -1e30, p)
    m2 = jnp.max(p2, axis=1, keepdims=True)
    i2 = jnp.min(jnp.where(p2 == m2, iota, E), axis=1, keepdims=True)
    s = m1 + m2
    idx_ref[...] = jnp.concatenate([i1, i2], axis=1)
    w_ref[...] = jnp.concatenate([m1 / s, m2 / s], axis=1)


def _ffn_kernel(te_ref, x_ref, wgu_ref, wd_ref, y_ref):
    gu = jnp.dot(x_ref[...], wgu_ref[0], preferred_element_type=jnp.float32)
    g = gu[:, :F]
    u = gu[:, F:]
    h = g * jax.nn.sigmoid(g) * u
    y_ref[...] = jnp.dot(h, wd_ref[0], preferred_element_type=jnp.float32)


_sc_mesh = plsc.VectorSubcoreMesh(core_axis_name="c", subcore_axis_name="s")


@functools.partial(
    pl.kernel,
    mesh=_sc_mesh,
    out_type=jax.ShapeDtypeStruct((NG, D), jnp.float32),
    scratch_types=[
        pltpu.VMEM((GB,), jnp.int32),
        pltpu.VMEM((GB,), jnp.int32),
        pltpu.VMEM((GB, D), jnp.float32),
        pltpu.SemaphoreType.DMA,
        pltpu.SemaphoreType.DMA,
    ],
)
def _sc_dispatch(hs_hbm, tok_hbm, slot_hbm, xs_hbm, tok_v, slot_v, rows_v,
                 sem_g, sem_s):
    wid = lax.axis_index("s") * 2 + lax.axis_index("c")
    base = wid * GB
    pltpu.sync_copy(tok_hbm.at[pl.ds(base, GB)], tok_v)
    pltpu.sync_copy(slot_hbm.at[pl.ds(base, GB)], slot_v)
    pltpu.async_copy(hs_hbm.at[tok_v], rows_v, sem_g).wait()
    pltpu.async_copy(rows_v, xs_hbm.at[slot_v], sem_s).wait()


@functools.partial(
    pl.kernel,
    mesh=_sc_mesh,
    out_type=jax.ShapeDtypeStruct((T, D), jnp.float32),
    scratch_types=[
        pltpu.VMEM((CB,), jnp.int32),
        pltpu.VMEM((CB,), jnp.int32),
        pltpu.VMEM((CB, D), jnp.float32),
        pltpu.VMEM((CB, D), jnp.float32),
        pltpu.VMEM((CB * LANES,), jnp.float32),
        pltpu.VMEM((CB * LANES,), jnp.float32),
        pltpu.SemaphoreType.DMA,
        pltpu.SemaphoreType.DMA,
    ],
)
def _sc_combine(ys_hbm, sa_hbm, sb_hbm, wa_hbm, wb_hbm, out_hbm,
                sa_v, sb_v, ya_v, yb_v, wa_v, wb_v, sem_a, sem_b):
    wid = lax.axis_index("s") * 2 + lax.axis_index("c")
    base = wid * CB
    pltpu.sync_copy(sa_hbm.at[pl.ds(base, CB)], sa_v)
    pltpu.sync_copy(sb_hbm.at[pl.ds(base, CB)], sb_v)
    pltpu.sync_copy(wa_hbm.at[pl.ds(base * LANES, CB * LANES)], wa_v)
    pltpu.sync_copy(wb_hbm.at[pl.ds(base * LANES, CB * LANES)], wb_v)
    ga = pltpu.async_copy(ys_hbm.at[sa_v], ya_v, sem_a)
    gb = pltpu.async_copy(ys_hbm.at[sb_v], yb_v, sem_b)
    ga.wait()
    gb.wait()

    def row_body(r, carry):
        wa = wa_v[pl.ds(r * LANES, LANES)]
        wb = wb_v[pl.ds(r * LANES, LANES)]
        for c in range(DC):
            ya = ya_v[r, pl.ds(c * LANES, LANES)]
            yb = yb_v[r, pl.ds(c * LANES, LANES)]
            ya_v[r, pl.ds(c * LANES, LANES)] = wa * ya + wb * yb
        return carry

    lax.fori_loop(0, CB, row_body, 0)
    pltpu.sync_copy(ya_v, out_hbm.at[pl.ds(base, CB)])


def kernel(hidden_states, W_gate, W_gate_up, W_down, num_global_tokens,
           max_num_tokens_per_gpu):
    hs = hidden_states
    topk_idx, topk_w = pl.pallas_call(
        _router_kernel,
        out_shape=(
            jax.ShapeDtypeStruct((T, K), jnp.int32),
            jax.ShapeDtypeStruct((T, K), jnp.float32),
        ),
    )(hs, W_gate)

    e_flat = topk_idx.reshape(-1)
    t_flat = jnp.repeat(jnp.arange(T, dtype=jnp.int32), K)
    onehot = e_flat[:, None] == jnp.arange(E, dtype=jnp.int32)[None, :]
    csum = jnp.cumsum(onehot.astype(jnp.int32), axis=0)
    counts = csum[-1]
    rank = jnp.sum(jnp.where(onehot, csum - 1, 0), axis=1)
    padded = ((counts + TM - 1) // TM) * TM
    pad_end = jnp.cumsum(padded)
    pad_start = pad_end - padded
    slot = (pad_start[e_flat] + rank).astype(jnp.int32)
    total = pad_end[-1]
    tile_start = jnp.arange(N_TILES, dtype=jnp.int32) * TM
    te = jnp.searchsorted(pad_end, jnp.minimum(tile_start, total - 1),
                          side="right").astype(jnp.int32)
    tile_expert = jnp.minimum(te, E - 1)

    x_sorted = _sc_dispatch(hs, t_flat, slot)

    grid_spec = pltpu.PrefetchScalarGridSpec(
        num_scalar_prefetch=1,
        grid=(N_TILES,),
        in_specs=[
            pl.BlockSpec((TM, D), lambda i, te: (i, 0)),
            pl.BlockSpec((1, D, 2 * F), lambda i, te: (te[i], 0, 0)),
            pl.BlockSpec((1, F, D), lambda i, te: (te[i], 0, 0)),
        ],
        out_specs=pl.BlockSpec((TM, D), lambda i, te: (i, 0)),
    )
    y_sorted = pl.pallas_call(
        _ffn_kernel,
        grid_spec=grid_spec,
        out_shape=jax.ShapeDtypeStruct((NG, D), jnp.float32),
    )(tile_expert, x_sorted, W_gate_up, W_down)

    inv_slot = slot.reshape(T, K)
    wa_b = jnp.broadcast_to(topk_w[:, 0:1], (T, LANES)).reshape(T * LANES)
    wb_b = jnp.broadcast_to(topk_w[:, 1:2], (T, LANES)).reshape(T * LANES)
    out = _sc_combine(y_sorted, inv_slot[:, 0], inv_slot[:, 1], wa_b, wb_b)
    return out
